# baseline (speedup 1.0000x reference)
import jax
import jax.numpy as jnp
from jax import lax
from jax.experimental import pallas as pl
from jax.experimental.pallas import tpu as pltpu
from jax.experimental.pallas import tpu_sc as plsc

_SUP = 300
_EPS = 0.001
_ROW = 2 * _SUP + 1
_N = 65536

_NC = 2
_NS = 16
_NW = _NC * _NS
_EPW = _N // _NW
_C = 24
_NCHUNK = _ROW // _C
_NVREG = _EPW // 16


def _prep_kernel(x_ref, s_ref, p_ref, a_ref, b_ref):
    x = x_ref[:]
    enc = jnp.sign(x) * (jnp.sqrt(jnp.abs(x) + 1.0) - 1.0) + _EPS * x
    enc = jnp.clip(enc, -float(_SUP), float(_SUP))
    fl = jnp.floor(enc)
    rem = enc - fl
    fli = fl.astype(jnp.int32)
    top = fli >= _SUP
    s_ref[:] = enc
    p_ref[:] = jnp.where(top, 2 * _SUP - 1, fli + _SUP)
    a_ref[:] = jnp.where(top, 0.0, 1.0 - rem)
    b_ref[:] = jnp.where(top, 1.0, rem)


def _sc_expand(p_hbm, a_hbm, b_hbm, out_hbm, p_v, a_v, b_v, bufs, insem, outsem):
    wid = lax.axis_index("s") * _NC + lax.axis_index("c")
    base = wid * _EPW
    pltpu.async_copy(p_hbm.at[pl.ds(base, _EPW)], p_v, insem)
    pltpu.async_copy(a_hbm.at[pl.ds(base, _EPW)], a_v, insem)
    pltpu.async_copy(b_hbm.at[pl.ds(base, _EPW)], b_v, insem).wait()
    pltpu.make_async_copy(a_hbm.at[pl.ds(base, _EPW)], a_v, insem).wait()
    pltpu.make_async_copy(p_hbm.at[pl.ds(base, _EPW)], p_v, insem).wait()

    zeros16 = jnp.zeros((16,), jnp.float32)
    lane = lax.iota(jnp.int32, 16)

    def _zero_row(r, _):
        def _zero_chunk(c, _):
            for b in range(2):
                bufs[b][r, pl.ds(c * 16, 16)] = zeros16
            return 0

        return lax.fori_loop(0, _EPW // 16, _zero_chunk, 0)

    lax.fori_loop(0, _C, _zero_row, 0)

    def _sweep(slot, c0, c0_old):
        buf = bufs[slot]

        cu = jnp.uint32(_C)

        def body(jo, _):
            for jj in range(4):
                j = jo * 4 + jj
                col = lane + j * 16
                p16 = p_v[pl.ds(j * 16, 16)]
                q16 = p16 + 1
                ro_a = p16 - c0_old
                ro_b = q16 - c0_old
                m_oa = ro_a.astype(jnp.uint32) < cu
                m_ob = ro_b.astype(jnp.uint32) < cu
                plsc.store_scatter(buf, [ro_a, col], zeros16, mask=m_oa)
                plsc.store_scatter(buf, [ro_b, col], zeros16, mask=m_ob)
                r_a = p16 - c0
                r_b = q16 - c0
                m_a = r_a.astype(jnp.uint32) < cu
                m_b = r_b.astype(jnp.uint32) < cu
                plsc.store_scatter(buf, [r_a, col], a_v[pl.ds(j * 16, 16)], mask=m_a)
                plsc.store_scatter(buf, [r_b, col], b_v[pl.ds(j * 16, 16)], mask=m_b)
            return 0

        lax.fori_loop(0, _NVREG // 4, body, 0)

    def _start_dma(slot, c0):
        pltpu.make_async_copy(
            bufs[slot], out_hbm.at[pl.ds(c0, _C), pl.ds(base, _EPW)], outsem
        ).start()

    def _wait_dma():
        pltpu.make_async_copy(
            bufs[0], out_hbm.at[pl.ds(0, _C), pl.ds(base, _EPW)], outsem
        ).wait()

    far = jnp.int32(-1 << 20)

    _sweep(0, jnp.int32(0), far)
    _start_dma(0, jnp.int32(0))
    _sweep(1, jnp.int32(_C), far)
    _start_dma(1, jnp.int32(_C))

    def _main(o, _):
        for s in range(2):
            k = 2 + o * 2 + s
            c0 = k * _C
            _wait_dma()
            _sweep(s, c0, c0 - 2 * _C)
            _start_dma(s, c0)
        return 0

    lax.fori_loop(0, (_NCHUNK - 2) // 2, _main, 0)
    k_last = _NCHUNK - 1
    _wait_dma()
    _sweep(0, jnp.int32(k_last * _C), jnp.int32((k_last - 2) * _C))
    _start_dma(0, jnp.int32(k_last * _C))

    _wait_dma()

    def _last_body(j, _):
        col = lane + j * 16
        p16 = p_v[pl.ds(j * 16, 16)]
        q16 = p16 + 1
        c0_old = jnp.int32((k_last - 1) * _C)
        m_oa = (p16 >= c0_old) & (p16 < c0_old + _C)
        m_ob = (q16 >= c0_old) & (q16 < c0_old + _C)
        plsc.store_scatter(bufs[1], [p16 - c0_old, col], zeros16, mask=m_oa)
        plsc.store_scatter(bufs[1], [q16 - c0_old, col], zeros16, mask=m_ob)
        m_b = q16 == 2 * _SUP
        plsc.store_scatter(
            bufs[1], [q16 - 2 * _SUP, col], b_v[pl.ds(j * 16, 16)], mask=m_b
        )
        return 0

    lax.fori_loop(0, _NVREG, _last_body, 0)
    pltpu.make_async_copy(
        bufs[1].at[pl.ds(0, 1)], out_hbm.at[pl.ds(2 * _SUP, 1), pl.ds(base, _EPW)],
        outsem,
    ).start()

    _wait_dma()
    pltpu.make_async_copy(
        bufs[1].at[pl.ds(0, 1)], out_hbm.at[pl.ds(2 * _SUP, 1), pl.ds(base, _EPW)],
        outsem,
    ).wait()


@jax.jit
def kernel(x):
    n = x.shape[0]
    x2 = x.reshape(512, 128)
    enc_s, p, a, b = pl.pallas_call(
        _prep_kernel,
        out_shape=[
            jax.ShapeDtypeStruct((512, 128), jnp.float32),
            jax.ShapeDtypeStruct((512, 128), jnp.int32),
            jax.ShapeDtypeStruct((512, 128), jnp.float32),
            jax.ShapeDtypeStruct((512, 128), jnp.float32),
        ],
    )(x2)

    sc = pl.kernel(
        _sc_expand,
        out_type=jax.ShapeDtypeStruct((_ROW, n), jnp.float32),
        mesh=plsc.VectorSubcoreMesh(core_axis_name="c", subcore_axis_name="s"),
        scratch_types=[
            pltpu.VMEM((_EPW,), jnp.int32),
            pltpu.VMEM((_EPW,), jnp.float32),
            pltpu.VMEM((_EPW,), jnp.float32),
            [pltpu.VMEM((_C, _EPW), jnp.float32) for _ in range(2)],
            pltpu.SemaphoreType.DMA,
            pltpu.SemaphoreType.DMA,
        ],
        compiler_params=pltpu.CompilerParams(needs_layout_passes=False),
    )
    enc_v_t = sc(p.reshape(n), a.reshape(n), b.reshape(n))
    return (enc_s.reshape(n), jnp.transpose(enc_v_t))


# speedup vs baseline: 1.1164x; 1.1164x over previous
import jax
import jax.numpy as jnp
from jax import lax
from jax.experimental import pallas as pl
from jax.experimental.pallas import tpu as pltpu
from jax.experimental.pallas import tpu_sc as plsc

_SUP = 300
_EPS = 0.001
_ROW = 2 * _SUP + 1
_N = 65536

_NC = 2
_NS = 16
_NW = _NC * _NS
_EPW = _N // _NW
_C = 24
_NCHUNK = _ROW // _C
_NVREG = _EPW // 16


def _prep_kernel(x_ref, s_ref, p_ref, a_ref, b_ref):
    x = x_ref[:]
    enc = jnp.sign(x) * (jnp.sqrt(jnp.abs(x) + 1.0) - 1.0) + _EPS * x
    enc = jnp.clip(enc, -float(_SUP), float(_SUP))
    fl = jnp.floor(enc)
    rem = enc - fl
    fli = fl.astype(jnp.int32)
    top = fli >= _SUP
    s_ref[:] = enc
    p_ref[:] = jnp.where(top, 2 * _SUP - 1, fli + _SUP)
    a_ref[:] = jnp.where(top, 0.0, 1.0 - rem)
    b_ref[:] = jnp.where(top, 1.0, rem)


def _sc_expand(p_hbm, a_hbm, b_hbm, out_hbm, p_v, a_v, b_v, bufs, insem, outsem):
    wid = lax.axis_index("s") * _NC + lax.axis_index("c")
    base = wid * _EPW
    pltpu.async_copy(p_hbm.at[pl.ds(base, _EPW)], p_v, insem)
    pltpu.async_copy(a_hbm.at[pl.ds(base, _EPW)], a_v, insem)
    pltpu.async_copy(b_hbm.at[pl.ds(base, _EPW)], b_v, insem)

    zeros16 = jnp.zeros((16,), jnp.float32)
    lane = lax.iota(jnp.int32, 16)

    def _zero_row(r, _):
        def _zero_chunk(c, _):
            for b in range(2):
                for cc in range(8):
                    bufs[b][r, pl.ds((c * 8 + cc) * 16, 16)] = zeros16
            return 0

        return lax.fori_loop(0, _EPW // 128, _zero_chunk, 0)

    lax.fori_loop(0, _C, _zero_row, 0)
    pltpu.make_async_copy(b_hbm.at[pl.ds(base, _EPW)], b_v, insem).wait()
    pltpu.make_async_copy(a_hbm.at[pl.ds(base, _EPW)], a_v, insem).wait()
    pltpu.make_async_copy(p_hbm.at[pl.ds(base, _EPW)], p_v, insem).wait()

    def _sweep(slot, c0, c0_old):
        buf = bufs[slot]

        cu = jnp.uint32(_C)

        def body(jo, _):
            for jj in range(4):
                j = jo * 4 + jj
                col = lane + j * 16
                p16 = p_v[pl.ds(j * 16, 16)]
                q16 = p16 + 1
                ro_a = p16 - c0_old
                ro_b = q16 - c0_old
                m_oa = ro_a.astype(jnp.uint32) < cu
                m_ob = ro_b.astype(jnp.uint32) < cu
                plsc.store_scatter(buf, [ro_a, col], zeros16, mask=m_oa)
                plsc.store_scatter(buf, [ro_b, col], zeros16, mask=m_ob)
                r_a = p16 - c0
                r_b = q16 - c0
                m_a = r_a.astype(jnp.uint32) < cu
                m_b = r_b.astype(jnp.uint32) < cu
                plsc.store_scatter(buf, [r_a, col], a_v[pl.ds(j * 16, 16)], mask=m_a)
                plsc.store_scatter(buf, [r_b, col], b_v[pl.ds(j * 16, 16)], mask=m_b)
            return 0

        lax.fori_loop(0, _NVREG // 4, body, 0)

    def _start_dma(slot, c0):
        pltpu.make_async_copy(
            bufs[slot], out_hbm.at[pl.ds(c0, _C), pl.ds(base, _EPW)], outsem
        ).start()

    def _wait_dma():
        pltpu.make_async_copy(
            bufs[0], out_hbm.at[pl.ds(0, _C), pl.ds(base, _EPW)], outsem
        ).wait()

    far = jnp.int32(-1 << 20)

    _sweep(0, jnp.int32(0), far)
    _start_dma(0, jnp.int32(0))
    _sweep(1, jnp.int32(_C), far)
    _start_dma(1, jnp.int32(_C))

    def _main(o, _):
        for s in range(2):
            k = 2 + o * 2 + s
            c0 = k * _C
            _wait_dma()
            _sweep(s, c0, c0 - 2 * _C)
            _start_dma(s, c0)
        return 0

    lax.fori_loop(0, (_NCHUNK - 2) // 2, _main, 0)
    k_last = _NCHUNK - 1
    _wait_dma()
    _sweep(0, jnp.int32(k_last * _C), jnp.int32((k_last - 2) * _C))
    _start_dma(0, jnp.int32(k_last * _C))

    _wait_dma()

    def _last_body(j, _):
        col = lane + j * 16
        p16 = p_v[pl.ds(j * 16, 16)]
        q16 = p16 + 1
        c0_old = jnp.int32((k_last - 1) * _C)
        m_oa = (p16 >= c0_old) & (p16 < c0_old + _C)
        m_ob = (q16 >= c0_old) & (q16 < c0_old + _C)
        plsc.store_scatter(bufs[1], [p16 - c0_old, col], zeros16, mask=m_oa)
        plsc.store_scatter(bufs[1], [q16 - c0_old, col], zeros16, mask=m_ob)
        m_b = q16 == 2 * _SUP
        plsc.store_scatter(
            bufs[1], [q16 - 2 * _SUP, col], b_v[pl.ds(j * 16, 16)], mask=m_b
        )
        return 0

    lax.fori_loop(0, _NVREG, _last_body, 0)
    pltpu.make_async_copy(
        bufs[1].at[pl.ds(0, 1)], out_hbm.at[pl.ds(2 * _SUP, 1), pl.ds(base, _EPW)],
        outsem,
    ).start()

    _wait_dma()
    pltpu.make_async_copy(
        bufs[1].at[pl.ds(0, 1)], out_hbm.at[pl.ds(2 * _SUP, 1), pl.ds(base, _EPW)],
        outsem,
    ).wait()


@jax.jit
def kernel(x):
    n = x.shape[0]
    x2 = x.reshape(512, 128)
    enc_s, p, a, b = pl.pallas_call(
        _prep_kernel,
        out_shape=[
            jax.ShapeDtypeStruct((512, 128), jnp.float32),
            jax.ShapeDtypeStruct((512, 128), jnp.int32),
            jax.ShapeDtypeStruct((512, 128), jnp.float32),
            jax.ShapeDtypeStruct((512, 128), jnp.float32),
        ],
    )(x2)

    sc = pl.kernel(
        _sc_expand,
        out_type=jax.ShapeDtypeStruct((_ROW, n), jnp.float32),
        mesh=plsc.VectorSubcoreMesh(core_axis_name="c", subcore_axis_name="s"),
        scratch_types=[
            pltpu.VMEM((_EPW,), jnp.int32),
            pltpu.VMEM((_EPW,), jnp.float32),
            pltpu.VMEM((_EPW,), jnp.float32),
            [pltpu.VMEM((_C, _EPW), jnp.float32) for _ in range(2)],
            pltpu.SemaphoreType.DMA,
            pltpu.SemaphoreType.DMA,
        ],
        compiler_params=pltpu.CompilerParams(needs_layout_passes=False),
    )
    enc_v_t = sc(p.reshape(n), a.reshape(n), b.reshape(n))
    return (enc_s.reshape(n), jnp.transpose(enc_v_t))


# speedup vs baseline: 1.2281x; 1.1000x over previous
import jax
import jax.numpy as jnp
from jax import lax
from jax.experimental import pallas as pl
from jax.experimental.pallas import tpu as pltpu
from jax.experimental.pallas import tpu_sc as plsc

_SUP = 300
_EPS = 0.001
_ROW = 2 * _SUP + 1
_N = 65536

_NC = 2
_NS = 16
_NW = _NC * _NS
_EPW = _N // _NW
_C = 24
_NCHUNK = _ROW // _C
_NVREG = _EPW // 16


def _prep_kernel(x_ref, s_ref, p_ref, a_ref):
    x = x_ref[:]
    enc = jnp.sign(x) * (jnp.sqrt(jnp.abs(x) + 1.0) - 1.0) + _EPS * x
    enc = jnp.clip(enc, -float(_SUP), float(_SUP))
    fl = jnp.floor(enc)
    rem = enc - fl
    fli = fl.astype(jnp.int32)
    top = fli >= _SUP
    s_ref[:] = enc
    p_ref[:] = jnp.where(top, 2 * _SUP - 1, fli + _SUP)
    a_ref[:] = jnp.where(top, 0.0, 1.0 - rem)


def _sc_expand(p_hbm, a_hbm, out_hbm, p_v, a_v, bufs, insem, outsem):
    wid = lax.axis_index("s") * _NC + lax.axis_index("c")
    base = wid * _EPW
    pltpu.async_copy(p_hbm.at[pl.ds(base, _EPW)], p_v, insem)
    pltpu.async_copy(a_hbm.at[pl.ds(base, _EPW)], a_v, insem)

    zeros16 = jnp.zeros((16,), jnp.float32)
    lane = lax.iota(jnp.int32, 16)

    def _zero_row(r, _):
        def _zero_chunk(c, _):
            for b in range(2):
                for cc in range(8):
                    bufs[b][r, pl.ds((c * 8 + cc) * 16, 16)] = zeros16
            return 0

        return lax.fori_loop(0, _EPW // 128, _zero_chunk, 0)

    lax.fori_loop(0, _C, _zero_row, 0)
    pltpu.make_async_copy(a_hbm.at[pl.ds(base, _EPW)], a_v, insem).wait()
    pltpu.make_async_copy(p_hbm.at[pl.ds(base, _EPW)], p_v, insem).wait()

    def _sweep(slot, c0, c0_old):
        buf = bufs[slot]

        cu = jnp.uint32(_C)

        def body(jo, _):
            for jj in range(4):
                j = jo * 4 + jj
                col = lane + j * 16
                p16 = p_v[pl.ds(j * 16, 16)]
                q16 = p16 + 1
                ro_a = p16 - c0_old
                ro_b = q16 - c0_old
                m_oa = ro_a.astype(jnp.uint32) < cu
                m_ob = ro_b.astype(jnp.uint32) < cu
                plsc.store_scatter(buf, [ro_a, col], zeros16, mask=m_oa)
                plsc.store_scatter(buf, [ro_b, col], zeros16, mask=m_ob)
                r_a = p16 - c0
                r_b = q16 - c0
                m_a = r_a.astype(jnp.uint32) < cu
                m_b = r_b.astype(jnp.uint32) < cu
                a16 = a_v[pl.ds(j * 16, 16)]
                plsc.store_scatter(buf, [r_a, col], a16, mask=m_a)
                plsc.store_scatter(buf, [r_b, col], 1.0 - a16, mask=m_b)
            return 0

        lax.fori_loop(0, _NVREG // 4, body, 0)

    def _start_dma(slot, c0):
        pltpu.make_async_copy(
            bufs[slot], out_hbm.at[pl.ds(c0, _C), pl.ds(base, _EPW)], outsem
        ).start()

    def _wait_dma():
        pltpu.make_async_copy(
            bufs[0], out_hbm.at[pl.ds(0, _C), pl.ds(base, _EPW)], outsem
        ).wait()

    far = jnp.int32(-1 << 20)

    _sweep(0, jnp.int32(0), far)
    _start_dma(0, jnp.int32(0))
    _sweep(1, jnp.int32(_C), far)
    _start_dma(1, jnp.int32(_C))

    def _main(o, _):
        for s in range(2):
            k = 2 + o * 2 + s
            c0 = k * _C
            _wait_dma()
            _sweep(s, c0, c0 - 2 * _C)
            _start_dma(s, c0)
        return 0

    lax.fori_loop(0, (_NCHUNK - 2) // 2, _main, 0)
    k_last = _NCHUNK - 1
    _wait_dma()
    _sweep(0, jnp.int32(k_last * _C), jnp.int32((k_last - 2) * _C))
    _start_dma(0, jnp.int32(k_last * _C))

    _wait_dma()

    def _last_body(j, _):
        col = lane + j * 16
        p16 = p_v[pl.ds(j * 16, 16)]
        q16 = p16 + 1
        c0_old = jnp.int32((k_last - 1) * _C)
        m_oa = (p16 >= c0_old) & (p16 < c0_old + _C)
        m_ob = (q16 >= c0_old) & (q16 < c0_old + _C)
        plsc.store_scatter(bufs[1], [p16 - c0_old, col], zeros16, mask=m_oa)
        plsc.store_scatter(bufs[1], [q16 - c0_old, col], zeros16, mask=m_ob)
        m_b = q16 == 2 * _SUP
        plsc.store_scatter(
            bufs[1], [q16 - 2 * _SUP, col], 1.0 - a_v[pl.ds(j * 16, 16)], mask=m_b
        )
        return 0

    lax.fori_loop(0, _NVREG, _last_body, 0)
    pltpu.make_async_copy(
        bufs[1].at[pl.ds(0, 1)], out_hbm.at[pl.ds(2 * _SUP, 1), pl.ds(base, _EPW)],
        outsem,
    ).start()

    _wait_dma()
    pltpu.make_async_copy(
        bufs[1].at[pl.ds(0, 1)], out_hbm.at[pl.ds(2 * _SUP, 1), pl.ds(base, _EPW)],
        outsem,
    ).wait()


@jax.jit
def kernel(x):
    n = x.shape[0]
    x2 = x.reshape(512, 128)
    enc_s, p, a = pl.pallas_call(
        _prep_kernel,
        out_shape=[
            jax.ShapeDtypeStruct((512, 128), jnp.float32),
            jax.ShapeDtypeStruct((512, 128), jnp.int32),
            jax.ShapeDtypeStruct((512, 128), jnp.float32),
        ],
    )(x2)

    sc = pl.kernel(
        _sc_expand,
        out_type=jax.ShapeDtypeStruct((_ROW, n), jnp.float32),
        mesh=plsc.VectorSubcoreMesh(core_axis_name="c", subcore_axis_name="s"),
        scratch_types=[
            pltpu.VMEM((_EPW,), jnp.int32),
            pltpu.VMEM((_EPW,), jnp.float32),
            [pltpu.VMEM((_C, _EPW), jnp.float32) for _ in range(2)],
            pltpu.SemaphoreType.DMA,
            pltpu.SemaphoreType.DMA,
        ],
        compiler_params=pltpu.CompilerParams(needs_layout_passes=False),
    )
    enc_v_t = sc(p.reshape(n), a.reshape(n))
    return (enc_s.reshape(n), jnp.transpose(enc_v_t))
